# Initial kernel scaffold; baseline (speedup 1.0000x reference)
import jax
import jax.numpy as jnp
from jax.experimental import pallas as pl


def kernel(embed, time_steps, noise, sqrt_alphas_cumprod, sqrt_one_minus_alphas_cumprod):
    raise NotImplementedError("write your pallas kernel here")


# speedup vs baseline: 3.3236x; 3.3236x over previous
import functools

import jax
import jax.numpy as jnp
from jax import lax
from jax.experimental import pallas as pl
from jax.experimental.pallas import tpu as pltpu
from jax.experimental.pallas import tpu_sc as plsc

B = 16384
D = 128
N_TAB = 1000
NC = 2
NS = 16
NW = NC * NS
RW = B // NW
CH = 128
NCHUNK = RW // CH
L = 16


def _body(embed_h, ts_h, noise_h, a_h, om_h, out_h,
          a_tab, om_tab, ts_v, ebuf, nbuf, obuf, ca_buf, com_buf):
    wid = lax.axis_index("s") * NC + lax.axis_index("c")
    base = wid * RW

    pltpu.sync_copy(a_h, a_tab)
    pltpu.sync_copy(om_h, om_tab)
    pltpu.sync_copy(ts_h.at[pl.ds(base, RW)], ts_v)

    def chunk_body(c, _):
        row0 = base + c * CH
        pltpu.sync_copy(embed_h.at[pl.ds(row0, CH)], ebuf)
        pltpu.sync_copy(noise_h.at[pl.ds(row0, CH)], nbuf)

        def group_body(g, _):
            idx = ts_v[pl.ds(c * CH + g * L, L)]
            ca = plsc.load_gather(a_tab, [idx])
            com = plsc.load_gather(om_tab, [idx])
            ca_buf[...] = ca
            com_buf[...] = com

            def row_body(j, _):
                r = g * L + j
                bidx = jnp.full((L,), j, dtype=jnp.int32)
                aj = plsc.load_gather(ca_buf, [bidx])
                omj = plsc.load_gather(com_buf, [bidx])
                for k in range(D // L):
                    sl = pl.ds(k * L, L)
                    obuf[r, sl] = aj * ebuf[r, sl] + omj * nbuf[r, sl]
                return ()

            lax.fori_loop(0, L, row_body, (), unroll=False)
            return ()

        lax.fori_loop(0, CH // L, group_body, (), unroll=False)
        pltpu.sync_copy(obuf, out_h.at[pl.ds(row0, CH)])
        return ()

    lax.fori_loop(0, NCHUNK, chunk_body, (), unroll=False)


@jax.jit
def _diffuse(embed, time_steps, noise, a_tab, om_tab):
    kfn = functools.partial(
        pl.kernel,
        out_type=jax.ShapeDtypeStruct((B, D), jnp.float32),
        mesh=plsc.VectorSubcoreMesh(core_axis_name="c", subcore_axis_name="s"),
        compiler_params=pltpu.CompilerParams(needs_layout_passes=False),
        scratch_types=[
            pltpu.VMEM((N_TAB,), jnp.float32),
            pltpu.VMEM((N_TAB,), jnp.float32),
            pltpu.VMEM((RW,), jnp.int32),
            pltpu.VMEM((CH, D), jnp.float32),
            pltpu.VMEM((CH, D), jnp.float32),
            pltpu.VMEM((CH, D), jnp.float32),
            pltpu.VMEM((L,), jnp.float32),
            pltpu.VMEM((L,), jnp.float32),
        ],
    )(_body)
    return kfn(embed, time_steps, noise, a_tab, om_tab)


def kernel(embed, time_steps, noise, sqrt_alphas_cumprod,
           sqrt_one_minus_alphas_cumprod):
    ts = time_steps.astype(jnp.int32)
    return _diffuse(embed, ts, noise, sqrt_alphas_cumprod,
                    sqrt_one_minus_alphas_cumprod)


# speedup vs baseline: 3.8659x; 1.1632x over previous
import functools

import jax
import jax.numpy as jnp
from jax import lax
from jax.experimental import pallas as pl
from jax.experimental.pallas import tpu as pltpu
from jax.experimental.pallas import tpu_sc as plsc

B = 16384
D = 128
N_TAB = 1000
NC = 2
NS = 16
NW = NC * NS
RW = B // NW
CH = 128
NCHUNK = RW // CH
L = 16


def _body(embed_h, ts_h, noise_h, a_h, om_h, out_h,
          a_tab, om_tab, ts_v,
          eb0, eb1, nb0, nb1, ob0, ob1,
          ca_buf, com_buf,
          es0, es1, ns0, ns1, os0, os1):
    wid = lax.axis_index("s") * NC + lax.axis_index("c")
    base = wid * RW

    ebufs = (eb0, eb1)
    nbufs = (nb0, nb1)
    obufs = (ob0, ob1)
    esems = (es0, es1)
    nsems = (ns0, ns1)
    osems = (os0, os1)

    def start_in(c):
        bsel = c & 1
        ecp = pltpu.async_copy(embed_h.at[pl.ds(base + c * CH, CH)],
                               ebufs[bsel], esems[bsel])
        ncp = pltpu.async_copy(noise_h.at[pl.ds(base + c * CH, CH)],
                               nbufs[bsel], nsems[bsel])
        return ecp, ncp

    in_cp = [None, None]
    out_cp = [None, None]
    in_cp[0] = start_in(0)

    pltpu.sync_copy(a_h, a_tab)
    pltpu.sync_copy(om_h, om_tab)
    pltpu.sync_copy(ts_h.at[pl.ds(base, RW)], ts_v)

    def compute_chunk(c, ebuf, nbuf, obuf):
        def group_body(g, _):
            idx = ts_v[pl.ds(c * CH + g * L, L)]
            ca = plsc.load_gather(a_tab, [idx])
            com = plsc.load_gather(om_tab, [idx])
            ca_buf[...] = ca
            com_buf[...] = com

            def row_body(j, _):
                r = g * L + j
                bidx = jnp.full((L,), j, dtype=jnp.int32)
                aj = plsc.load_gather(ca_buf, [bidx])
                omj = plsc.load_gather(com_buf, [bidx])
                for k in range(D // L):
                    sl = pl.ds(k * L, L)
                    obuf[r, sl] = aj * ebuf[r, sl] + omj * nbuf[r, sl]
                return ()

            lax.fori_loop(0, L, row_body, (), unroll=4)
            return ()

        lax.fori_loop(0, CH // L, group_body, (), unroll=False)

    for c in range(NCHUNK):
        bsel = c & 1
        if c + 1 < NCHUNK:
            in_cp[1 - bsel] = start_in(c + 1)
        ecp, ncp = in_cp[bsel]
        ecp.wait()
        ncp.wait()
        if c >= 2:
            out_cp[bsel].wait()
        compute_chunk(c, ebufs[bsel], nbufs[bsel], obufs[bsel])
        out_cp[bsel] = pltpu.async_copy(
            obufs[bsel], out_h.at[pl.ds(base + c * CH, CH)], osems[bsel])

    out_cp[(NCHUNK - 2) & 1].wait()
    out_cp[(NCHUNK - 1) & 1].wait()


@jax.jit
def _diffuse(embed, time_steps, noise, a_tab, om_tab):
    kfn = functools.partial(
        pl.kernel,
        out_type=jax.ShapeDtypeStruct((B, D), jnp.float32),
        mesh=plsc.VectorSubcoreMesh(core_axis_name="c", subcore_axis_name="s"),
        compiler_params=pltpu.CompilerParams(needs_layout_passes=False),
        scratch_types=[
            pltpu.VMEM((N_TAB,), jnp.float32),
            pltpu.VMEM((N_TAB,), jnp.float32),
            pltpu.VMEM((RW,), jnp.int32),
            pltpu.VMEM((CH, D), jnp.float32),
            pltpu.VMEM((CH, D), jnp.float32),
            pltpu.VMEM((CH, D), jnp.float32),
            pltpu.VMEM((CH, D), jnp.float32),
            pltpu.VMEM((CH, D), jnp.float32),
            pltpu.VMEM((CH, D), jnp.float32),
            pltpu.VMEM((L,), jnp.float32),
            pltpu.VMEM((L,), jnp.float32),
            pltpu.SemaphoreType.DMA,
            pltpu.SemaphoreType.DMA,
            pltpu.SemaphoreType.DMA,
            pltpu.SemaphoreType.DMA,
            pltpu.SemaphoreType.DMA,
            pltpu.SemaphoreType.DMA,
        ],
    )(_body)
    return kfn(embed, time_steps, noise, a_tab, om_tab)


def kernel(embed, time_steps, noise, sqrt_alphas_cumprod,
           sqrt_one_minus_alphas_cumprod):
    ts = time_steps.astype(jnp.int32)
    return _diffuse(embed, ts, noise, sqrt_alphas_cumprod,
                    sqrt_one_minus_alphas_cumprod)


# speedup vs baseline: 6.0331x; 1.5606x over previous
import functools

import jax
import jax.numpy as jnp
from jax import lax
from jax.experimental import pallas as pl
from jax.experimental.pallas import tpu as pltpu
from jax.experimental.pallas import tpu_sc as plsc

B = 16384
D = 128
N_TAB = 1000
NC = 2
NS = 16
NW = NC * NS
RW = B // NW
CH = 128
NCHUNK = RW // CH
L = 16


def _body(embed_h, ts_h, noise_h, a_h, om_h, out_h,
          a_tab, om_tab, ts_v,
          eb0, eb1, nb0, nb1, ob0, ob1,
          es0, es1, ns0, ns1, os0, os1):
    wid = lax.axis_index("s") * NC + lax.axis_index("c")
    base = wid * RW

    ebufs = (eb0, eb1)
    nbufs = (nb0, nb1)
    obufs = (ob0, ob1)
    esems = (es0, es1)
    nsems = (ns0, ns1)
    osems = (os0, os1)

    def start_in(c):
        bsel = c & 1
        ecp = pltpu.async_copy(embed_h.at[pl.ds(base + c * CH, CH)],
                               ebufs[bsel], esems[bsel])
        ncp = pltpu.async_copy(noise_h.at[pl.ds(base + c * CH, CH)],
                               nbufs[bsel], nsems[bsel])
        return ecp, ncp

    in_cp = [None, None]
    out_cp = [None, None]
    in_cp[0] = start_in(0)

    pltpu.sync_copy(a_h, a_tab)
    pltpu.sync_copy(om_h, om_tab)
    pltpu.sync_copy(ts_h.at[pl.ds(base, RW)], ts_v)

    def compute_chunk(c, ebuf, nbuf, obuf):
        @plsc.parallel_loop(0, CH, unroll=4)
        def rows(r):
            bidx = jnp.full((L,), c * CH + r, dtype=jnp.int32)
            t_b = plsc.load_gather(ts_v, [bidx])
            aj = plsc.load_gather(a_tab, [t_b])
            omj = plsc.load_gather(om_tab, [t_b])
            for k in range(D // L):
                sl = pl.ds(k * L, L)
                obuf[r, sl] = aj * ebuf[r, sl] + omj * nbuf[r, sl]

    for c in range(NCHUNK):
        bsel = c & 1
        if c + 1 < NCHUNK:
            in_cp[1 - bsel] = start_in(c + 1)
        ecp, ncp = in_cp[bsel]
        ecp.wait()
        ncp.wait()
        if c >= 2:
            out_cp[bsel].wait()
        compute_chunk(c, ebufs[bsel], nbufs[bsel], obufs[bsel])
        out_cp[bsel] = pltpu.async_copy(
            obufs[bsel], out_h.at[pl.ds(base + c * CH, CH)], osems[bsel])

    out_cp[(NCHUNK - 2) & 1].wait()
    out_cp[(NCHUNK - 1) & 1].wait()


@jax.jit
def _diffuse(embed, time_steps, noise, a_tab, om_tab):
    kfn = functools.partial(
        pl.kernel,
        out_type=jax.ShapeDtypeStruct((B, D), jnp.float32),
        mesh=plsc.VectorSubcoreMesh(core_axis_name="c", subcore_axis_name="s"),
        compiler_params=pltpu.CompilerParams(needs_layout_passes=False),
        scratch_types=[
            pltpu.VMEM((N_TAB,), jnp.float32),
            pltpu.VMEM((N_TAB,), jnp.float32),
            pltpu.VMEM((RW,), jnp.int32),
            pltpu.VMEM((CH, D), jnp.float32),
            pltpu.VMEM((CH, D), jnp.float32),
            pltpu.VMEM((CH, D), jnp.float32),
            pltpu.VMEM((CH, D), jnp.float32),
            pltpu.VMEM((CH, D), jnp.float32),
            pltpu.VMEM((CH, D), jnp.float32),
            pltpu.SemaphoreType.DMA,
            pltpu.SemaphoreType.DMA,
            pltpu.SemaphoreType.DMA,
            pltpu.SemaphoreType.DMA,
            pltpu.SemaphoreType.DMA,
            pltpu.SemaphoreType.DMA,
        ],
    )(_body)
    return kfn(embed, time_steps, noise, a_tab, om_tab)


def kernel(embed, time_steps, noise, sqrt_alphas_cumprod,
           sqrt_one_minus_alphas_cumprod):
    ts = time_steps.astype(jnp.int32)
    return _diffuse(embed, ts, noise, sqrt_alphas_cumprod,
                    sqrt_one_minus_alphas_cumprod)


# speedup vs baseline: 6.0408x; 1.0013x over previous
import functools

import jax
import jax.numpy as jnp
from jax import lax
from jax.experimental import pallas as pl
from jax.experimental.pallas import tpu as pltpu
from jax.experimental.pallas import tpu_sc as plsc

B = 16384
D = 128
N_TAB = 1000
NC = 2
NS = 16
NW = NC * NS
RW = B // NW
CH = 128
NCHUNK = RW // CH
L = 16


def _body(embed_h, ts_h, noise_h, a_h, om_h, out_h,
          a_tab, om_tab, ts_v,
          eb0, eb1, nb0, nb1, ob0, ob1,
          es0, es1, ns0, ns1, os0, os1):
    wid = lax.axis_index("s") * NC + lax.axis_index("c")
    base = wid * RW

    ebufs = (eb0, eb1)
    nbufs = (nb0, nb1)
    obufs = (ob0, ob1)
    esems = (es0, es1)
    nsems = (ns0, ns1)
    osems = (os0, os1)

    def start_in(c):
        bsel = c & 1
        ecp = pltpu.async_copy(embed_h.at[pl.ds(base + c * CH, CH)],
                               ebufs[bsel], esems[bsel])
        ncp = pltpu.async_copy(noise_h.at[pl.ds(base + c * CH, CH)],
                               nbufs[bsel], nsems[bsel])
        return ecp, ncp

    in_cp = [None, None]
    out_cp = [None, None]
    in_cp[0] = start_in(0)

    pltpu.sync_copy(a_h, a_tab)
    pltpu.sync_copy(om_h, om_tab)
    pltpu.sync_copy(ts_h.at[pl.ds(base, RW)], ts_v)

    def compute_chunk(c, ebuf, nbuf, obuf):
        @plsc.parallel_loop(0, CH, unroll=4)
        def rows(r):
            bidx = jnp.full((L,), c * CH + r, dtype=jnp.int32)
            t_b = plsc.load_gather(ts_v, [bidx])
            aj = plsc.load_gather(a_tab, [t_b])
            omj = plsc.load_gather(om_tab, [t_b])
            for k in range(D // L):
                sl = pl.ds(k * L, L)
                obuf[r, sl] = aj * ebuf[r, sl] + omj * nbuf[r, sl]

    for c in range(NCHUNK):
        bsel = c & 1
        if c + 1 < NCHUNK:
            in_cp[1 - bsel] = start_in(c + 1)
        ecp, ncp = in_cp[bsel]
        ecp.wait()
        ncp.wait()
        if c >= 2:
            out_cp[bsel].wait()
        compute_chunk(c, ebufs[bsel], nbufs[bsel], obufs[bsel])
        out_cp[bsel] = pltpu.async_copy(
            obufs[bsel], out_h.at[pl.ds(base + c * CH, CH)], osems[bsel])

    out_cp[(NCHUNK - 2) & 1].wait()
    out_cp[(NCHUNK - 1) & 1].wait()


@jax.jit
def _diffuse(embed, time_steps, noise, a_tab, om_tab):
    kfn = functools.partial(
        pl.kernel,
        out_type=jax.ShapeDtypeStruct((B, D), jnp.float32),
        mesh=plsc.VectorSubcoreMesh(core_axis_name="c", subcore_axis_name="s"),
        compiler_params=pltpu.CompilerParams(
            needs_layout_passes=False,
            skip_device_barrier=True,
            disable_bounds_checks=True,
            disable_semaphore_checks=True,
        ),
        scratch_types=[
            pltpu.VMEM((N_TAB,), jnp.float32),
            pltpu.VMEM((N_TAB,), jnp.float32),
            pltpu.VMEM((RW,), jnp.int32),
            pltpu.VMEM((CH, D), jnp.float32),
            pltpu.VMEM((CH, D), jnp.float32),
            pltpu.VMEM((CH, D), jnp.float32),
            pltpu.VMEM((CH, D), jnp.float32),
            pltpu.VMEM((CH, D), jnp.float32),
            pltpu.VMEM((CH, D), jnp.float32),
            pltpu.SemaphoreType.DMA,
            pltpu.SemaphoreType.DMA,
            pltpu.SemaphoreType.DMA,
            pltpu.SemaphoreType.DMA,
            pltpu.SemaphoreType.DMA,
            pltpu.SemaphoreType.DMA,
        ],
    )(_body)
    return kfn(embed, time_steps, noise, a_tab, om_tab)


def kernel(embed, time_steps, noise, sqrt_alphas_cumprod,
           sqrt_one_minus_alphas_cumprod):
    ts = time_steps.astype(jnp.int32)
    return _diffuse(embed, ts, noise, sqrt_alphas_cumprod,
                    sqrt_one_minus_alphas_cumprod)
